# baseline (speedup 1.0000x reference)
import functools

import jax
import jax.numpy as jnp
from jax import lax
from jax.experimental import pallas as pl
from jax.experimental.pallas import tpu as pltpu
from jax.experimental.pallas import tpu_sc as plsc

N_NODES = 10000
N_PAD = 10240
N_EDGES = 320000
D = 128

NC = 2
NS = 16
NW = NC * NS
EPW = N_EDGES // NW
CHUNK = 40
NCHUNK = EPW // CHUNK
CH2 = 128
EPW_PAD = 10240
NCH2 = EPW_PAD // CH2
NCH2H = NCH2 // 2
RPT = N_PAD // NS
ZR = 64

_mesh = plsc.VectorSubcoreMesh(core_axis_name="c", subcore_axis_name="s")


def _agg_body(y_hbm, src_hbm, dst_hbm, agg_out, sidx_a, sidx_b, didx_a,
              didx_b, rows_a, rows_b, acc, gsem, isem):
  cc = lax.axis_index("c")
  ss = lax.axis_index("s")
  wid = ss * NC + cc
  base = wid * EPW

  zero16 = jnp.zeros((16,), jnp.float32)

  @pl.loop(0, CHUNK)
  def _zero_fill(r):
    @pl.loop(0, D // 16)
    def _(j):
      rows_a[r, pl.ds(j * 16, 16)] = zero16

  @pl.loop(0, RPT // CHUNK)
  def _zero_acc(r):
    pltpu.sync_copy(rows_a, acc.at[pl.ds(ss * RPT + r * CHUNK, CHUNK)])

  plsc.subcore_barrier()

  def _idx_start(i, sbuf, dbuf):
    pltpu.async_copy(src_hbm.at[pl.ds(base + i * CHUNK, CHUNK)], sbuf, isem)
    pltpu.async_copy(dst_hbm.at[pl.ds(base + i * CHUNK, CHUNK)], dbuf, isem)

  def _idx_wait(i, sbuf, dbuf):
    pltpu.make_async_copy(src_hbm.at[pl.ds(base + i * CHUNK, CHUNK)], sbuf,
                          isem).wait()
    pltpu.make_async_copy(dst_hbm.at[pl.ds(base + i * CHUNK, CHUNK)], dbuf,
                          isem).wait()

  def _gather_start(sbuf, buf):
    pltpu.async_copy(y_hbm.at[sbuf], buf, gsem)

  def _gather_wait(sbuf, buf):
    pltpu.make_async_copy(y_hbm.at[sbuf], buf, gsem).wait()

  def _scat(dbuf, buf):
    pltpu.sync_copy(buf, acc.at[dbuf], add=True)

  pltpu.sync_copy(src_hbm.at[pl.ds(base, CHUNK)], sidx_a)
  pltpu.sync_copy(dst_hbm.at[pl.ds(base, CHUNK)], didx_a)
  _gather_start(sidx_a, rows_a)

  @pl.loop(0, NCHUNK - 2, step=2)
  def _edges(i):
    _idx_start(i + 1, sidx_b, didx_b)
    _gather_wait(sidx_a, rows_a)
    _scat(didx_a, rows_a)
    _idx_wait(i + 1, sidx_b, didx_b)
    _gather_start(sidx_b, rows_b)
    _idx_start(i + 2, sidx_a, didx_a)
    _gather_wait(sidx_b, rows_b)
    _scat(didx_b, rows_b)
    _idx_wait(i + 2, sidx_a, didx_a)
    _gather_start(sidx_a, rows_a)

  _idx_start(NCHUNK - 1, sidx_b, didx_b)
  _gather_wait(sidx_a, rows_a)
  _scat(didx_a, rows_a)
  _idx_wait(NCHUNK - 1, sidx_b, didx_b)
  _gather_start(sidx_b, rows_b)
  _gather_wait(sidx_b, rows_b)
  _scat(didx_b, rows_b)

  plsc.subcore_barrier()

  pltpu.sync_copy(acc.at[pl.ds(ss * RPT, RPT)],
                  agg_out.at[cc, pl.ds(ss * RPT, RPT)])


_agg = pl.kernel(
    _agg_body,
    out_type=[jax.ShapeDtypeStruct((NC, N_PAD, D), jnp.float32)],
    mesh=_mesh,
    scratch_types=[
        pltpu.VMEM((CHUNK,), jnp.int32),
        pltpu.VMEM((CHUNK,), jnp.int32),
        pltpu.VMEM((CHUNK,), jnp.int32),
        pltpu.VMEM((CHUNK,), jnp.int32),
        pltpu.VMEM((CHUNK, D), jnp.float32),
        pltpu.VMEM((CHUNK, D), jnp.float32),
        pltpu.VMEM_SHARED((N_PAD, D), jnp.float32),
        pltpu.SemaphoreType.DMA,
        pltpu.SemaphoreType.DMA,
    ],
)


def _deg_body(dst3_hbm, deg_out, didx_all, ones, dacc):
  cc = lax.axis_index("c")
  ss = lax.axis_index("s")
  wid = ss * NC + cc

  zero16 = jnp.zeros((16,), jnp.float32)
  one16 = jnp.ones((16,), jnp.float32)

  @pl.loop(0, CH2)
  def _zero_fill(r):
    @pl.loop(0, D // 16)
    def _(j):
      ones[r, pl.ds(j * 16, 16)] = zero16

  @pl.loop(0, RPT // CH2)
  def _zero_acc(r):
    pltpu.sync_copy(ones, dacc.at[pl.ds(ss * RPT + r * CH2, CH2)])

  @pl.loop(0, CH2)
  def _ones_fill(r):
    @pl.loop(0, D // 16)
    def _(j):
      ones[r, pl.ds(j * 16, 16)] = one16

  plsc.subcore_barrier()

  for h in range(2):
    pltpu.sync_copy(dst3_hbm.at[wid, pl.ds(h * NCH2H, NCH2H)], didx_all)

    @pl.loop(0, NCH2H)
    def _edges(i):
      pltpu.sync_copy(ones, dacc.at[didx_all.at[i]], add=True)

  plsc.subcore_barrier()

  pltpu.sync_copy(dacc.at[pl.ds(ss * RPT, RPT)],
                  deg_out.at[cc, pl.ds(ss * RPT, RPT)])


_deg = pl.kernel(
    _deg_body,
    out_type=[jax.ShapeDtypeStruct((NC, N_PAD, D), jnp.float32)],
    mesh=_mesh,
    scratch_types=[
        pltpu.VMEM((NCH2H, CH2), jnp.int32),
        pltpu.VMEM((CH2, D), jnp.float32),
        pltpu.VMEM_SHARED((N_PAD, D), jnp.float32),
    ],
)


_R = 2000
_GRID = N_NODES // _R


def _dot(a, b):
  return jnp.dot(a, b, preferred_element_type=jnp.float32,
                 precision=lax.Precision.HIGHEST)


def _pre_body(x_ref, ws_ref, wn_ref, b_ref, z_ref, y_ref):
  xb = x_ref[...]
  z_ref[...] = _dot(xb, ws_ref[...]) + b_ref[...]
  y_ref[...] = _dot(xb, wn_ref[...])


def _mix_body(z_ref, agg_ref, dg_ref, ws_ref, wn_ref, b_ref, z2_ref, y2_ref):
  a = agg_ref[0] + agg_ref[1]
  d = dg_ref[0, :, 0:1] + dg_ref[1, :, 0:1]
  inv = 1.0 / jnp.maximum(d, 1.0)
  h = jnp.maximum(z_ref[...] + a * inv, 0.0)
  z2_ref[...] = _dot(h, ws_ref[...]) + b_ref[...]
  y2_ref[...] = _dot(h, wn_ref[...])


def _fin_body(z_ref, agg_ref, dg_ref, wc_ref, bc_ref, o_ref):
  a = agg_ref[0] + agg_ref[1]
  d = dg_ref[0, :, 0:1] + dg_ref[1, :, 0:1]
  inv = 1.0 / jnp.maximum(d, 1.0)
  h = jnp.maximum(z_ref[...] + a * inv, 0.0)
  o_ref[...] = _dot(h, wc_ref[...]) + bc_ref[...]


def _row_spec(r, d):
  return pl.BlockSpec((r, d), lambda i: (i, 0))


def _part_spec(r, d):
  return pl.BlockSpec((NC, r, d), lambda i: (0, i, 0))


def _deg_spec(r):
  return pl.BlockSpec((NC, r, D), lambda i: (0, i, 0))


def _full_spec(a, b):
  return pl.BlockSpec((a, b), lambda i: (0, 0))


def _pre_call(x, ws, wn, b):
  return pl.pallas_call(
      _pre_body,
      grid=(_GRID,),
      in_specs=[_row_spec(_R, D), _full_spec(D, D), _full_spec(D, D),
                _full_spec(1, D)],
      out_specs=[_row_spec(_R, D), _row_spec(_R, D)],
      out_shape=[jax.ShapeDtypeStruct((N_NODES, D), jnp.float32),
                 jax.ShapeDtypeStruct((N_PAD, D), jnp.float32)],
  )(x, ws, wn, b)


def _mix_call(z, agg, dg, ws, wn, b):
  return pl.pallas_call(
      _mix_body,
      grid=(_GRID,),
      in_specs=[_row_spec(_R, D), _part_spec(_R, D), _deg_spec(_R),
                _full_spec(D, D), _full_spec(D, D), _full_spec(1, D)],
      out_specs=[_row_spec(_R, D), _row_spec(_R, D)],
      out_shape=[jax.ShapeDtypeStruct((N_NODES, D), jnp.float32),
                 jax.ShapeDtypeStruct((N_PAD, D), jnp.float32)],
  )(z, agg, dg, ws, wn, b)


def _fin_call(z, agg, dg, wc, bc):
  return pl.pallas_call(
      _fin_body,
      grid=(_GRID,),
      in_specs=[_row_spec(_R, D), _part_spec(_R, D), _deg_spec(_R),
                _full_spec(D, D), _full_spec(1, D)],
      out_specs=_row_spec(_R, D),
      out_shape=jax.ShapeDtypeStruct((N_NODES, D), jnp.float32),
  )(z, agg, dg, wc, bc)


@jax.jit
def kernel(x, edge_index, W1_self, W1_neigh, b1, W2_self, W2_neigh, b2, Wc,
           bc):
  ei = edge_index.astype(jnp.int32)
  src = ei[0]
  dst = ei[1]
  npad = EPW_PAD - EPW
  pad_dst = jnp.full((NW, npad), N_PAD - 1, jnp.int32)
  dst3 = jnp.concatenate([dst.reshape(NW, EPW), pad_dst],
                         axis=1).reshape(NW, NCH2, CH2)

  (dg,) = _deg(dst3)
  z1, y1 = _pre_call(x, W1_self, W1_neigh, b1.reshape(1, D))
  (agg1,) = _agg(y1, src, dst)
  z2, y2 = _mix_call(z1, agg1, dg, W2_self, W2_neigh, b2.reshape(1, D))
  (agg2,) = _agg(y2, src, dst)

  wc_pad = jnp.zeros((D, D), jnp.float32).at[:, :40].set(Wc)
  bc_pad = jnp.zeros((1, D), jnp.float32).at[0, :40].set(bc)
  out = _fin_call(z2, agg2, dg, wc_pad, bc_pad)
  return out[:, :40]


# speedup vs baseline: 1.5855x; 1.5855x over previous
import functools

import jax
import jax.numpy as jnp
from jax import lax
from jax.experimental import pallas as pl
from jax.experimental.pallas import tpu as pltpu
from jax.experimental.pallas import tpu_sc as plsc

N_NODES = 10000
N_PAD = 10240
N_EDGES = 320000
D = 128

NC = 2
NS = 16
NW = NC * NS
EPW = N_EDGES // NW
CHUNK = 80
NCHUNK = EPW // CHUNK
CH2 = 128
EPW_PAD = 10240
NCH2 = EPW_PAD // CH2
NCH2H = NCH2 // 2
RPT = N_PAD // NS
ZR = 64

_mesh = plsc.VectorSubcoreMesh(core_axis_name="c", subcore_axis_name="s")


def _agg_body(y_hbm, src_hbm, dst_hbm, agg_out, sidx_a, sidx_b, didx_a,
              didx_b, rows_a, rows_b, acc, gsem, isem, ssem):
  cc = lax.axis_index("c")
  ss = lax.axis_index("s")
  wid = ss * NC + cc
  base = wid * EPW

  zero16 = jnp.zeros((16,), jnp.float32)

  @pl.loop(0, CHUNK)
  def _zero_fill(r):
    @pl.loop(0, D // 16)
    def _(j):
      rows_a[r, pl.ds(j * 16, 16)] = zero16

  @pl.loop(0, RPT // CHUNK)
  def _zero_acc(r):
    pltpu.sync_copy(rows_a, acc.at[pl.ds(ss * RPT + r * CHUNK, CHUNK)])

  plsc.subcore_barrier()

  def _idx_start(i, sbuf, dbuf):
    pltpu.async_copy(src_hbm.at[pl.ds(base + i * CHUNK, CHUNK)], sbuf, isem)
    pltpu.async_copy(dst_hbm.at[pl.ds(base + i * CHUNK, CHUNK)], dbuf, isem)

  def _idx_wait(i, sbuf, dbuf):
    pltpu.make_async_copy(src_hbm.at[pl.ds(base + i * CHUNK, CHUNK)], sbuf,
                          isem).wait()
    pltpu.make_async_copy(dst_hbm.at[pl.ds(base + i * CHUNK, CHUNK)], dbuf,
                          isem).wait()

  def _gather_start(sbuf, buf):
    pltpu.async_copy(y_hbm.at[sbuf], buf, gsem)

  def _gather_wait(sbuf, buf):
    pltpu.make_async_copy(y_hbm.at[sbuf], buf, gsem).wait()

  def _scat_start(dbuf, buf):
    pltpu.async_copy(buf, acc.at[dbuf], ssem, add=True)

  def _scat_wait(dbuf, buf):
    pltpu.make_async_copy(buf, acc.at[dbuf], ssem).wait()

  pltpu.sync_copy(src_hbm.at[pl.ds(base, CHUNK)], sidx_a)
  pltpu.sync_copy(dst_hbm.at[pl.ds(base, CHUNK)], didx_a)
  _gather_start(sidx_a, rows_a)

  @pl.loop(0, NCHUNK - 1, step=2)
  def _edges(i):
    _idx_start(i + 1, sidx_b, didx_b)
    _gather_wait(sidx_a, rows_a)
    _scat_start(didx_a, rows_a)
    _idx_wait(i + 1, sidx_b, didx_b)
    _gather_start(sidx_b, rows_b)
    _idx_start(i + 2, sidx_a, didx_a)
    _gather_wait(sidx_b, rows_b)
    _scat_start(didx_b, rows_b)
    _scat_wait(didx_a, rows_a)
    _idx_wait(i + 2, sidx_a, didx_a)
    _gather_start(sidx_a, rows_a)
    _scat_wait(didx_b, rows_b)

  _gather_wait(sidx_a, rows_a)
  _scat_start(didx_a, rows_a)
  _scat_wait(didx_a, rows_a)

  plsc.subcore_barrier()

  pltpu.sync_copy(acc.at[pl.ds(ss * RPT, RPT)],
                  agg_out.at[cc, pl.ds(ss * RPT, RPT)])


_agg = pl.kernel(
    _agg_body,
    out_type=[jax.ShapeDtypeStruct((NC, N_PAD, D), jnp.float32)],
    mesh=_mesh,
    scratch_types=[
        pltpu.VMEM((CHUNK,), jnp.int32),
        pltpu.VMEM((CHUNK,), jnp.int32),
        pltpu.VMEM((CHUNK,), jnp.int32),
        pltpu.VMEM((CHUNK,), jnp.int32),
        pltpu.VMEM((CHUNK, D), jnp.float32),
        pltpu.VMEM((CHUNK, D), jnp.float32),
        pltpu.VMEM_SHARED((N_PAD, D), jnp.float32),
        pltpu.SemaphoreType.DMA,
        pltpu.SemaphoreType.DMA,
        pltpu.SemaphoreType.DMA,
    ],
)


def _deg_body(dst3_hbm, deg_out, didx_all, ones, dacc, dsem):
  cc = lax.axis_index("c")
  ss = lax.axis_index("s")
  wid = ss * NC + cc

  zero16 = jnp.zeros((16,), jnp.float32)
  one16 = jnp.ones((16,), jnp.float32)

  @pl.loop(0, CH2)
  def _zero_fill(r):
    @pl.loop(0, D // 16)
    def _(j):
      ones[r, pl.ds(j * 16, 16)] = zero16

  @pl.loop(0, RPT // CH2)
  def _zero_acc(r):
    pltpu.sync_copy(ones, dacc.at[pl.ds(ss * RPT + r * CH2, CH2)])

  @pl.loop(0, CH2)
  def _ones_fill(r):
    @pl.loop(0, D // 16)
    def _(j):
      ones[r, pl.ds(j * 16, 16)] = one16

  plsc.subcore_barrier()

  for h in range(2):
    pltpu.sync_copy(dst3_hbm.at[wid, pl.ds(h * NCH2H, NCH2H)], didx_all)

    @pl.loop(0, NCH2H, step=8)
    def _edges(i):
      for k in range(8):
        pltpu.async_copy(ones, dacc.at[didx_all.at[i + k]], dsem, add=True)
      for k in range(8):
        pltpu.make_async_copy(ones, dacc.at[didx_all.at[i + k]],
                              dsem).wait()

  plsc.subcore_barrier()

  pltpu.sync_copy(dacc.at[pl.ds(ss * RPT, RPT)],
                  deg_out.at[cc, pl.ds(ss * RPT, RPT)])


_deg = pl.kernel(
    _deg_body,
    out_type=[jax.ShapeDtypeStruct((NC, N_PAD, D), jnp.float32)],
    mesh=_mesh,
    scratch_types=[
        pltpu.VMEM((NCH2H, CH2), jnp.int32),
        pltpu.VMEM((CH2, D), jnp.float32),
        pltpu.VMEM_SHARED((N_PAD, D), jnp.float32),
        pltpu.SemaphoreType.DMA,
    ],
)


_R = 2000
_GRID = N_NODES // _R


def _dot(a, b):
  return jnp.dot(a, b, preferred_element_type=jnp.float32,
                 precision=lax.Precision.HIGHEST)


def _pre_body(x_ref, ws_ref, wn_ref, b_ref, z_ref, y_ref):
  xb = x_ref[...]
  z_ref[...] = _dot(xb, ws_ref[...]) + b_ref[...]
  y_ref[...] = _dot(xb, wn_ref[...])


def _mix_body(z_ref, agg_ref, dg_ref, ws_ref, wn_ref, b_ref, z2_ref, y2_ref):
  a = agg_ref[0] + agg_ref[1]
  d = dg_ref[0, :, 0:1] + dg_ref[1, :, 0:1]
  inv = 1.0 / jnp.maximum(d, 1.0)
  h = jnp.maximum(z_ref[...] + a * inv, 0.0)
  z2_ref[...] = _dot(h, ws_ref[...]) + b_ref[...]
  y2_ref[...] = _dot(h, wn_ref[...])


def _fin_body(z_ref, agg_ref, dg_ref, wc_ref, bc_ref, o_ref):
  a = agg_ref[0] + agg_ref[1]
  d = dg_ref[0, :, 0:1] + dg_ref[1, :, 0:1]
  inv = 1.0 / jnp.maximum(d, 1.0)
  h = jnp.maximum(z_ref[...] + a * inv, 0.0)
  o_ref[...] = _dot(h, wc_ref[...]) + bc_ref[...]


def _row_spec(r, d):
  return pl.BlockSpec((r, d), lambda i: (i, 0))


def _part_spec(r, d):
  return pl.BlockSpec((NC, r, d), lambda i: (0, i, 0))


def _deg_spec(r):
  return pl.BlockSpec((NC, r, D), lambda i: (0, i, 0))


def _full_spec(a, b):
  return pl.BlockSpec((a, b), lambda i: (0, 0))


def _pre_call(x, ws, wn, b):
  return pl.pallas_call(
      _pre_body,
      grid=(_GRID,),
      in_specs=[_row_spec(_R, D), _full_spec(D, D), _full_spec(D, D),
                _full_spec(1, D)],
      out_specs=[_row_spec(_R, D), _row_spec(_R, D)],
      out_shape=[jax.ShapeDtypeStruct((N_NODES, D), jnp.float32),
                 jax.ShapeDtypeStruct((N_PAD, D), jnp.float32)],
  )(x, ws, wn, b)


def _mix_call(z, agg, dg, ws, wn, b):
  return pl.pallas_call(
      _mix_body,
      grid=(_GRID,),
      in_specs=[_row_spec(_R, D), _part_spec(_R, D), _deg_spec(_R),
                _full_spec(D, D), _full_spec(D, D), _full_spec(1, D)],
      out_specs=[_row_spec(_R, D), _row_spec(_R, D)],
      out_shape=[jax.ShapeDtypeStruct((N_NODES, D), jnp.float32),
                 jax.ShapeDtypeStruct((N_PAD, D), jnp.float32)],
  )(z, agg, dg, ws, wn, b)


def _fin_call(z, agg, dg, wc, bc):
  return pl.pallas_call(
      _fin_body,
      grid=(_GRID,),
      in_specs=[_row_spec(_R, D), _part_spec(_R, D), _deg_spec(_R),
                _full_spec(D, D), _full_spec(1, D)],
      out_specs=_row_spec(_R, D),
      out_shape=jax.ShapeDtypeStruct((N_NODES, D), jnp.float32),
  )(z, agg, dg, wc, bc)


@jax.jit
def kernel(x, edge_index, W1_self, W1_neigh, b1, W2_self, W2_neigh, b2, Wc,
           bc):
  ei = edge_index.astype(jnp.int32)
  src = ei[0]
  dst = ei[1]
  npad = EPW_PAD - EPW
  pad_dst = jnp.full((NW, npad), N_PAD - 1, jnp.int32)
  dst3 = jnp.concatenate([dst.reshape(NW, EPW), pad_dst],
                         axis=1).reshape(NW, NCH2, CH2)

  (dg,) = _deg(dst3)
  z1, y1 = _pre_call(x, W1_self, W1_neigh, b1.reshape(1, D))
  (agg1,) = _agg(y1, src, dst)
  z2, y2 = _mix_call(z1, agg1, dg, W2_self, W2_neigh, b2.reshape(1, D))
  (agg2,) = _agg(y2, src, dst)

  wc_pad = jnp.zeros((D, D), jnp.float32).at[:, :40].set(Wc)
  bc_pad = jnp.zeros((1, D), jnp.float32).at[0, :40].set(bc)
  out = _fin_call(z2, agg2, dg, wc_pad, bc_pad)
  return out[:, :40]


# speedup vs baseline: 1.8567x; 1.1710x over previous
import functools

import jax
import jax.numpy as jnp
from jax import lax
from jax.experimental import pallas as pl
from jax.experimental.pallas import tpu as pltpu
from jax.experimental.pallas import tpu_sc as plsc

N_NODES = 10000
N_PAD = 10240
N_EDGES = 320000
D = 128

NC = 2
NS = 16
NW = NC * NS
EPW = N_EDGES // NW
CHUNK = 80
NCHUNK = EPW // CHUNK
CH2 = 128
EPW_PAD = 10240
NCH2 = EPW_PAD // CH2
NCH2H = NCH2 // 2
RPT = N_PAD // NS
ZR = 64

_mesh = plsc.VectorSubcoreMesh(core_axis_name="c", subcore_axis_name="s")


def _agg_body(y_hbm, src_hbm, dst_hbm, agg_out, sidx_a, sidx_b, sidx_c,
              sidx_d, didx_a, didx_b, didx_c, didx_d, rows_a, rows_b,
              rows_c, rows_d, acc, gsem, isem, ssem):
  cc = lax.axis_index("c")
  ss = lax.axis_index("s")
  wid = ss * NC + cc
  base = wid * EPW

  zero16 = jnp.zeros((16,), jnp.float32)

  @pl.loop(0, CHUNK)
  def _zero_fill(r):
    @pl.loop(0, D // 16)
    def _(j):
      rows_a[r, pl.ds(j * 16, 16)] = zero16

  @pl.loop(0, RPT // CHUNK)
  def _zero_acc(r):
    pltpu.sync_copy(rows_a, acc.at[pl.ds(ss * RPT + r * CHUNK, CHUNK)])

  plsc.subcore_barrier()

  def _idx_start(i, sbuf, dbuf):
    pltpu.async_copy(src_hbm.at[pl.ds(base + i * CHUNK, CHUNK)], sbuf, isem)
    pltpu.async_copy(dst_hbm.at[pl.ds(base + i * CHUNK, CHUNK)], dbuf, isem)

  def _idx_wait(i, sbuf, dbuf):
    pltpu.make_async_copy(src_hbm.at[pl.ds(base + i * CHUNK, CHUNK)], sbuf,
                          isem).wait()
    pltpu.make_async_copy(dst_hbm.at[pl.ds(base + i * CHUNK, CHUNK)], dbuf,
                          isem).wait()

  def _gather_start(sbuf, buf):
    pltpu.async_copy(y_hbm.at[sbuf], buf, gsem)

  def _gather_wait(sbuf, buf):
    pltpu.make_async_copy(y_hbm.at[sbuf], buf, gsem).wait()

  def _scat_start(dbuf, buf):
    pltpu.async_copy(buf, acc.at[dbuf], ssem, add=True)

  def _scat_wait(dbuf, buf):
    pltpu.make_async_copy(buf, acc.at[dbuf], ssem).wait()

  bufs = [(sidx_a, didx_a, rows_a), (sidx_b, didx_b, rows_b),
          (sidx_c, didx_c, rows_c), (sidx_d, didx_d, rows_d)]

  for b, (sb, db, rb) in enumerate(bufs):
    pltpu.sync_copy(src_hbm.at[pl.ds(base + b * CHUNK, CHUNK)], sb)
    pltpu.sync_copy(dst_hbm.at[pl.ds(base + b * CHUNK, CHUNK)], db)
    _gather_start(sb, rb)

  @pl.loop(0, NCHUNK - 5, step=4)
  def _edges(i):
    for b, (sb, db, rb) in enumerate(bufs):
      _gather_wait(sb, rb)
      _scat_start(db, rb)
    for b, (sb, db, rb) in enumerate(bufs):
      _scat_wait(db, rb)
      _idx_start(i + 4 + b, sb, db)
    for b, (sb, db, rb) in enumerate(bufs):
      _idx_wait(i + 4 + b, sb, db)
      _gather_start(sb, rb)

  for b, (sb, db, rb) in enumerate(bufs):
    _gather_wait(sb, rb)
    _scat_start(db, rb)
  _scat_wait(didx_a, rows_a)
  pltpu.sync_copy(src_hbm.at[pl.ds(base + (NCHUNK - 1) * CHUNK, CHUNK)],
                  sidx_a)
  pltpu.sync_copy(dst_hbm.at[pl.ds(base + (NCHUNK - 1) * CHUNK, CHUNK)],
                  didx_a)
  _gather_start(sidx_a, rows_a)
  _gather_wait(sidx_a, rows_a)
  _scat_start(didx_a, rows_a)
  _scat_wait(didx_a, rows_a)
  for _, (sb, db, rb) in enumerate(bufs[1:]):
    _scat_wait(db, rb)

  plsc.subcore_barrier()

  pltpu.sync_copy(acc.at[pl.ds(ss * RPT, RPT)],
                  agg_out.at[cc, pl.ds(ss * RPT, RPT)])


_agg = pl.kernel(
    _agg_body,
    out_type=[jax.ShapeDtypeStruct((NC, N_PAD, D), jnp.float32)],
    mesh=_mesh,
    scratch_types=(
        [pltpu.VMEM((CHUNK,), jnp.int32)] * 8 +
        [pltpu.VMEM((CHUNK, D), jnp.float32)] * 4 +
        [
            pltpu.VMEM_SHARED((N_PAD, D), jnp.float32),
            pltpu.SemaphoreType.DMA,
            pltpu.SemaphoreType.DMA,
            pltpu.SemaphoreType.DMA,
        ]),
)


def _deg_body(dst3_hbm, deg_out, didx_all, ones, dacc, dsem):
  cc = lax.axis_index("c")
  ss = lax.axis_index("s")
  wid = ss * NC + cc

  zero16 = jnp.zeros((16,), jnp.float32)
  one16 = jnp.ones((16,), jnp.float32)

  @pl.loop(0, CH2)
  def _zero_fill(r):
    @pl.loop(0, D // 16)
    def _(j):
      ones[r, pl.ds(j * 16, 16)] = zero16

  @pl.loop(0, RPT // CH2)
  def _zero_acc(r):
    pltpu.sync_copy(ones, dacc.at[pl.ds(ss * RPT + r * CH2, CH2)])

  @pl.loop(0, CH2)
  def _ones_fill(r):
    @pl.loop(0, D // 16)
    def _(j):
      ones[r, pl.ds(j * 16, 16)] = one16

  plsc.subcore_barrier()

  for h in range(2):
    pltpu.sync_copy(dst3_hbm.at[wid, pl.ds(h * NCH2H, NCH2H)], didx_all)

    @pl.loop(0, NCH2H, step=8)
    def _edges(i):
      for k in range(8):
        pltpu.async_copy(ones, dacc.at[didx_all.at[i + k]], dsem, add=True)
      for k in range(8):
        pltpu.make_async_copy(ones, dacc.at[didx_all.at[i + k]],
                              dsem).wait()

  plsc.subcore_barrier()

  pltpu.sync_copy(dacc.at[pl.ds(ss * RPT, RPT)],
                  deg_out.at[cc, pl.ds(ss * RPT, RPT)])


_deg = pl.kernel(
    _deg_body,
    out_type=[jax.ShapeDtypeStruct((NC, N_PAD, D), jnp.float32)],
    mesh=_mesh,
    scratch_types=[
        pltpu.VMEM((NCH2H, CH2), jnp.int32),
        pltpu.VMEM((CH2, D), jnp.float32),
        pltpu.VMEM_SHARED((N_PAD, D), jnp.float32),
        pltpu.SemaphoreType.DMA,
    ],
)


_R = 2000
_GRID = N_NODES // _R


def _dot(a, b):
  return jnp.dot(a, b, preferred_element_type=jnp.float32,
                 precision=lax.Precision.HIGHEST)


def _pre_body(x_ref, ws_ref, wn_ref, b_ref, z_ref, y_ref):
  xb = x_ref[...]
  z_ref[...] = _dot(xb, ws_ref[...]) + b_ref[...]
  y_ref[...] = _dot(xb, wn_ref[...])


def _mix_body(z_ref, agg_ref, dg_ref, ws_ref, wn_ref, b_ref, z2_ref, y2_ref):
  a = agg_ref[0] + agg_ref[1]
  d = dg_ref[0, :, 0:1] + dg_ref[1, :, 0:1]
  inv = 1.0 / jnp.maximum(d, 1.0)
  h = jnp.maximum(z_ref[...] + a * inv, 0.0)
  z2_ref[...] = _dot(h, ws_ref[...]) + b_ref[...]
  y2_ref[...] = _dot(h, wn_ref[...])


def _fin_body(z_ref, agg_ref, dg_ref, wc_ref, bc_ref, o_ref):
  a = agg_ref[0] + agg_ref[1]
  d = dg_ref[0, :, 0:1] + dg_ref[1, :, 0:1]
  inv = 1.0 / jnp.maximum(d, 1.0)
  h = jnp.maximum(z_ref[...] + a * inv, 0.0)
  o_ref[...] = _dot(h, wc_ref[...]) + bc_ref[...]


def _row_spec(r, d):
  return pl.BlockSpec((r, d), lambda i: (i, 0))


def _part_spec(r, d):
  return pl.BlockSpec((NC, r, d), lambda i: (0, i, 0))


def _deg_spec(r):
  return pl.BlockSpec((NC, r, D), lambda i: (0, i, 0))


def _full_spec(a, b):
  return pl.BlockSpec((a, b), lambda i: (0, 0))


def _pre_call(x, ws, wn, b):
  return pl.pallas_call(
      _pre_body,
      grid=(_GRID,),
      in_specs=[_row_spec(_R, D), _full_spec(D, D), _full_spec(D, D),
                _full_spec(1, D)],
      out_specs=[_row_spec(_R, D), _row_spec(_R, D)],
      out_shape=[jax.ShapeDtypeStruct((N_NODES, D), jnp.float32),
                 jax.ShapeDtypeStruct((N_PAD, D), jnp.float32)],
  )(x, ws, wn, b)


def _mix_call(z, agg, dg, ws, wn, b):
  return pl.pallas_call(
      _mix_body,
      grid=(_GRID,),
      in_specs=[_row_spec(_R, D), _part_spec(_R, D), _deg_spec(_R),
                _full_spec(D, D), _full_spec(D, D), _full_spec(1, D)],
      out_specs=[_row_spec(_R, D), _row_spec(_R, D)],
      out_shape=[jax.ShapeDtypeStruct((N_NODES, D), jnp.float32),
                 jax.ShapeDtypeStruct((N_PAD, D), jnp.float32)],
  )(z, agg, dg, ws, wn, b)


def _fin_call(z, agg, dg, wc, bc):
  return pl.pallas_call(
      _fin_body,
      grid=(_GRID,),
      in_specs=[_row_spec(_R, D), _part_spec(_R, D), _deg_spec(_R),
                _full_spec(D, D), _full_spec(1, D)],
      out_specs=_row_spec(_R, D),
      out_shape=jax.ShapeDtypeStruct((N_NODES, D), jnp.float32),
  )(z, agg, dg, wc, bc)


@jax.jit
def kernel(x, edge_index, W1_self, W1_neigh, b1, W2_self, W2_neigh, b2, Wc,
           bc):
  ei = edge_index.astype(jnp.int32)
  src = ei[0]
  dst = ei[1]
  npad = EPW_PAD - EPW
  pad_dst = jnp.full((NW, npad), N_PAD - 1, jnp.int32)
  dst3 = jnp.concatenate([dst.reshape(NW, EPW), pad_dst],
                         axis=1).reshape(NW, NCH2, CH2)

  (dg,) = _deg(dst3)
  z1, y1 = _pre_call(x, W1_self, W1_neigh, b1.reshape(1, D))
  (agg1,) = _agg(y1, src, dst)
  z2, y2 = _mix_call(z1, agg1, dg, W2_self, W2_neigh, b2.reshape(1, D))
  (agg2,) = _agg(y2, src, dst)

  wc_pad = jnp.zeros((D, D), jnp.float32).at[:, :40].set(Wc)
  bc_pad = jnp.zeros((1, D), jnp.float32).at[0, :40].set(bc)
  out = _fin_call(z2, agg2, dg, wc_pad, bc_pad)
  return out[:, :40]
